# Initial kernel scaffold; baseline (speedup 1.0000x reference)
import jax
import jax.numpy as jnp
from jax.experimental import pallas as pl


def kernel(z, pos, batch, edge_index, emb, W_si0, r0_w1, r0_b1, r0_w2, r0_b2, r0_w3, r0_b3, W_si1, r1_w1, r1_b1, r1_w2, r1_b2, r1_w3, r1_b3):
    raise NotImplementedError("write your pallas kernel here")


# speedup vs baseline: 1.5747x; 1.5747x over previous
import functools

import jax
import jax.numpy as jnp
from jax import lax
from jax.experimental import pallas as pl
from jax.experimental.pallas import tpu as pltpu
from jax.experimental.pallas import tpu_sc as plsc

N = 50000
E = 800000
NGRAPH = 64
MUL = 30
NSH = 4
CUTOFF = 10.0
NBASIS = 40
RADH = 64
NUM_NEIGHBORS = 20

NP_ = 51200
EP = 819200
TILE_E = 2048
TILE_N = 1600
NWORK = 32
EPW = EP // NWORK
NACC = NP_ // 4

C_HALF = 0.5 ** 0.5
Y0 = 1.0 / (NUM_NEIGHBORS ** 0.5)
Y1 = (3.0 ** 0.5) / (NUM_NEIGHBORS ** 0.5)
BWIDTH = CUTOFF / NBASIS
BSTEP = CUTOFF / (NBASIS - 1)

_SC_MESH = dict(core_axis_name="c", subcore_axis_name="s")


def _swish(x):
    return x * (1.0 / (1.0 + jnp.exp(-x)))


def _sc_gather_compact(table, idx3d, width):
    mesh = plsc.VectorSubcoreMesh(**_SC_MESH)
    nchunk = EPW // 1024
    nslot = 128 // width
    crows = 256 // nslot

    def compact(b_v, c_v):
        for kslot in range(nslot):
            for seg in range(width // 16):
                @pl.loop(0, crows)
                def _(p):
                    c_v.at[p, pl.ds(kslot * width + seg * 16, 16)][...] = (
                        b_v.at[kslot * crows + p, pl.ds(seg * 16, 16)][...])

    @functools.partial(
        pl.kernel, mesh=mesh,
        out_type=jax.ShapeDtypeStruct((EP // 256, crows, 128), jnp.float32),
        scratch_types=[
            pltpu.VMEM((8, 128), jnp.int32),
            pltpu.VMEM((256, 128), jnp.float32),
            pltpu.VMEM((256, 128), jnp.float32),
            pltpu.VMEM((crows, 128), jnp.float32),
            pltpu.VMEM((crows, 128), jnp.float32),
            pltpu.SemaphoreType.DMA,
            pltpu.SemaphoreType.DMA,
            pltpu.SemaphoreType.DMA,
            pltpu.SemaphoreType.DMA,
            pltpu.SemaphoreType.DMA,
        ],
    )
    def k(table_hbm, i_hbm, o_hbm, i_v, b0, b1, c0, c1,
          sem_i, sem_g0, sem_g1, sem_o0, sem_o1):
        wid = lax.axis_index("s") * 2 + lax.axis_index("c")
        bufs = [(b0, c0, sem_g0, sem_o0), (b1, c1, sem_g1, sem_o1)]

        def fire(q):
            b_v, _, sem_g, _ = bufs[q % 2]
            return [pltpu.async_copy(table_hbm.at[i_v.at[q * 2 + j]],
                                     b_v.at[pl.ds(j * 128, 128)], sem_g)
                    for j in range(2)]

        @pl.loop(0, nchunk)
        def _(ci):
            pltpu.async_copy(i_hbm.at[wid * nchunk + ci], i_v, sem_i).wait()
            hs = fire(0)
            ws = [None, None]
            for q in range(4):
                b_v, c_v, _, sem_o = bufs[q % 2]
                hs_next = fire(q + 1) if q < 3 else []
                for h in hs:
                    h.wait()
                hs = hs_next
                if ws[q % 2] is not None:
                    ws[q % 2].wait()
                compact(b_v, c_v)
                ws[q % 2] = pltpu.async_copy(
                    c_v, o_hbm.at[(wid * nchunk + ci) * 4 + q], sem_o)
            for w in ws:
                w.wait()

    return k(table, idx3d)


def _sc_scatter_add(dstq_3d, msgp, zeros_src):
    mesh = plsc.VectorSubcoreMesh(**_SC_MESH)
    nchunk = EPW // 1024

    @functools.partial(
        pl.kernel, mesh=mesh,
        out_type=jax.ShapeDtypeStruct((2, NACC, 128), jnp.float32),
        scratch_types=[
            pltpu.VMEM((8, 128), jnp.int32),
            pltpu.VMEM((128, 128), jnp.float32),
            pltpu.VMEM_SHARED((NACC, 128), jnp.float32),
            pltpu.SemaphoreType.DMA,
            pltpu.SemaphoreType.DMA,
        ],
    )
    def k(d_hbm, m_hbm, z_hbm, o_hbm, i_v, b_v, acc, sem_a, sem_o):
        cid = lax.axis_index("c")
        sid = lax.axis_index("s")

        @pl.when(sid == 0)
        def _():
            pltpu.async_copy(z_hbm, acc, sem_a).wait()

        plsc.subcore_barrier()
        wchunk0 = (cid * 16 + sid) * nchunk

        @pl.loop(0, nchunk)
        def _(ci):
            pltpu.async_copy(d_hbm.at[wchunk0 + ci], i_v, sem_a).wait()
            for j in range(8):
                pltpu.async_copy(
                    m_hbm.at[(wchunk0 + ci) * 8 + j], b_v, sem_a).wait()
                pltpu.sync_copy(b_v, acc.at[i_v.at[j]], add=True)
        plsc.subcore_barrier()

        @pl.when(sid == 0)
        def _():
            pltpu.async_copy(acc, o_hbm.at[cid], sem_o).wait()

    return k(dstq_3d, msgp, zeros_src)


def _radial_block(r, w1, b1, w2, b2, w3p, b3p):
    centers = lax.broadcasted_iota(
        jnp.int32, (1, NBASIS), 1).astype(jnp.float32) * BSTEP
    d = (r - centers) * (1.0 / BWIDTH)
    basis = jnp.exp(-d * d)
    h = _swish(jnp.dot(basis, w1, preferred_element_type=jnp.float32) + b1)
    h = _swish(jnp.dot(h, w2, preferred_element_type=jnp.float32) + b2)
    return jnp.dot(h, w3p, preferred_element_type=jnp.float32) + b3p


def _unpack(arr, width):
    nslot = 128 // width
    parts = []
    for g in range(arr.shape[0]):
        blk = arr[g]
        for k in range(nslot):
            parts.append(blk[:, k * width:(k + 1) * width])
    return jnp.concatenate(parts, axis=0)


def _coeff(wp, shx, shy, shz):
    return (Y0 * wp[:, 0:32] + shx * wp[:, 32:64]
            + shy * wp[:, 64:96] + shz * wp[:, 96:128])


def _edge0_body(sg_ref, dg_ref, dstc_ref, emb_ref, w1_ref, b1_ref, w2_ref,
                b2_ref, w3p_ref, b3p_ref, msgp_ref, rsh_ref):
    sg = _unpack(sg_ref[...], 16)
    dg = _unpack(dg_ref[...], 16)
    d = sg[:, 0:3] - dg[:, 0:3]
    dx = d[:, 0:1]
    dy = d[:, 1:2]
    dz = d[:, 2:3]
    r = jnp.sqrt(dx * dx + dy * dy + dz * dz)
    inv = Y1 / (r + 1e-9)
    shx = dx * inv
    shy = dy * inv
    shz = dz * inv
    zsrc = sg[:, 3:4].astype(jnp.int32)
    oh = (zsrc == lax.broadcasted_iota(jnp.int32, (TILE_E, 128), 1))
    xj = jnp.dot(oh.astype(jnp.float32), emb_ref[...],
                 preferred_element_type=jnp.float32)
    wp = _radial_block(r, w1_ref[...], b1_ref[...], w2_ref[...], b2_ref[...],
                       w3p_ref[...], b3p_ref[...])
    msg = xj * _coeff(wp, shx, shy, shz)
    eid = (pl.program_id(0) * TILE_E
           + lax.broadcasted_iota(jnp.int32, (TILE_E, 1), 0))
    msg = jnp.where(eid < E, msg, 0.0)
    dband = lax.div(dstc_ref[...], NACC)
    gidx = lax.div(lax.broadcasted_iota(jnp.int32, (TILE_E, 128), 1), 32)
    tile4 = jnp.concatenate([msg, msg, msg, msg], axis=1)
    placed = jnp.where(gidx == dband, tile4, 0.0)
    msgp_ref[...] = placed.reshape(TILE_E // 128, 128, 128)
    rsh_ref[...] = jnp.concatenate(
        [r, shx, shy, shz, jnp.zeros((TILE_E, 4), jnp.float32)], axis=1)


def _edge1_body(rsh_ref, xg_ref, dg_ref, w1_ref, b1_ref, w2_ref, b2_ref,
                w3p_ref, b3p_ref, out_ref):
    rsh = rsh_ref[...]
    r = rsh[:, 0:1]
    wp = _radial_block(r, w1_ref[...], b1_ref[...], w2_ref[...], b2_ref[...],
                       w3p_ref[...], b3p_ref[...])
    coeff = _coeff(wp, rsh[:, 1:2], rsh[:, 2:3], rsh[:, 3:4])
    xg = _unpack(xg_ref[...], 32)
    m = jnp.sum(xg * coeff, axis=1, keepdims=True)
    eid = (pl.program_id(0) * TILE_E
           + lax.broadcasted_iota(jnp.int32, (TILE_E, 1), 0))
    m = jnp.where(eid < E, m, 0.0)
    gdst = _unpack(dg_ref[...], 16)[:, 4:5].astype(jnp.int32)
    oh = (gdst == lax.broadcasted_iota(jnp.int32, (TILE_E, NGRAPH), 1))
    contrib = jnp.sum(jnp.where(oh, m, 0.0), axis=0, keepdims=True)

    @pl.when(pl.program_id(0) == 0)
    def _():
        out_ref[...] = jnp.zeros_like(out_ref)

    out_ref[...] += contrib


def _node_body(p_ref, accA_ref, accB_ref, emb_ref, wsi_ref, h1f_ref, h1t_ref):
    zf = p_ref[:, 3:4].astype(jnp.int32)
    oh = (zf == lax.broadcasted_iota(jnp.int32, (TILE_N, 128), 1))
    h0 = jnp.dot(oh.astype(jnp.float32), emb_ref[...],
                 preferred_element_type=jnp.float32)
    s0 = jnp.dot(h0, wsi_ref[...], preferred_element_type=jnp.float32)
    p = accA_ref[0] + accB_ref[0]
    band = pl.program_id(0) // (NACC // TILE_N)
    agg = jnp.where(band == 0, p[:, 0:32],
                    jnp.where(band == 1, p[:, 32:64],
                              jnp.where(band == 2, p[:, 64:96],
                                        p[:, 96:128])))
    h1 = _swish(C_HALF * s0 + C_HALF * agg)
    h1f_ref[...] = h1
    h1t_ref[...] = jnp.concatenate(
        [h1, jnp.zeros((TILE_N, 96), jnp.float32)], axis=1)


def _final_body(h1_ref, b_ref, g1_ref, wsi_ref, out_ref):
    s1 = jnp.dot(h1_ref[...], wsi_ref[...], preferred_element_type=jnp.float32)
    val = s1[:, 0:1]
    oh = (b_ref[...] == lax.broadcasted_iota(jnp.int32, (TILE_N, NGRAPH), 1))
    contrib = jnp.sum(jnp.where(oh, val, 0.0), axis=0, keepdims=True)

    @pl.when(pl.program_id(0) == 0)
    def _():
        out_ref[...] = g1_ref[...]

    out_ref[...] += contrib

    @pl.when(pl.program_id(0) == (NP_ // TILE_N) - 1)
    def _():
        out_ref[...] = C_HALF * out_ref[...]


def _full(shape):
    return pl.BlockSpec(shape, lambda i: tuple(0 for _ in shape))


def _tc_edge0(srcg, dstg, dstc, emb_pad, w1, b1, w2, b2, w3p, b3p):
    return pl.pallas_call(
        _edge0_body,
        grid=(EP // TILE_E,),
        in_specs=[
            pl.BlockSpec((TILE_E // 256, 32, 128), lambda i: (i, 0, 0)),
            pl.BlockSpec((TILE_E // 256, 32, 128), lambda i: (i, 0, 0)),
            pl.BlockSpec((TILE_E, 1), lambda i: (i, 0)),
            _full((128, 32)), _full((NBASIS, RADH)), _full((1, RADH)),
            _full((RADH, RADH)), _full((1, RADH)),
            _full((RADH, 128)), _full((1, 128)),
        ],
        out_specs=[
            pl.BlockSpec((TILE_E // 128, 128, 128), lambda i: (i, 0, 0)),
            pl.BlockSpec((TILE_E, 8), lambda i: (i, 0)),
        ],
        out_shape=[
            jax.ShapeDtypeStruct((EP // 128, 128, 128), jnp.float32),
            jax.ShapeDtypeStruct((EP, 8), jnp.float32),
        ],
    )(srcg, dstg, dstc, emb_pad, w1, b1, w2, b2, w3p, b3p)


def _tc_edge1(rsh, xj1g, dstg, w1, b1, w2, b2, w3p, b3p):
    return pl.pallas_call(
        _edge1_body,
        grid=(EP // TILE_E,),
        in_specs=[
            pl.BlockSpec((TILE_E, 8), lambda i: (i, 0)),
            pl.BlockSpec((TILE_E // 256, 64, 128), lambda i: (i, 0, 0)),
            pl.BlockSpec((TILE_E // 256, 32, 128), lambda i: (i, 0, 0)),
            _full((NBASIS, RADH)), _full((1, RADH)),
            _full((RADH, RADH)), _full((1, RADH)),
            _full((RADH, 128)), _full((1, 128)),
        ],
        out_specs=pl.BlockSpec((1, NGRAPH), lambda i: (0, 0)),
        out_shape=jax.ShapeDtypeStruct((1, NGRAPH), jnp.float32),
    )(rsh, xj1g, dstg, w1, b1, w2, b2, w3p, b3p)


def _tc_node(ptab, acc0, emb_pad, wsi0p):
    return pl.pallas_call(
        _node_body,
        grid=(NP_ // TILE_N,),
        in_specs=[
            pl.BlockSpec((TILE_N, 128), lambda i: (i, 0)),
            pl.BlockSpec((1, TILE_N, 128),
                         lambda i: (0, i % (NACC // TILE_N), 0)),
            pl.BlockSpec((1, TILE_N, 128),
                         lambda i: (1, i % (NACC // TILE_N), 0)),
            _full((128, 32)), _full((32, 32)),
        ],
        out_specs=[
            pl.BlockSpec((TILE_N, 32), lambda i: (i, 0)),
            pl.BlockSpec((TILE_N, 128), lambda i: (i, 0)),
        ],
        out_shape=[
            jax.ShapeDtypeStruct((NP_, 32), jnp.float32),
            jax.ShapeDtypeStruct((NP_, 128), jnp.float32),
        ],
    )(ptab, acc0, acc0, emb_pad, wsi0p)


def _tc_final(h1f, batch2d, gsum1, wsi1p):
    return pl.pallas_call(
        _final_body,
        grid=(NP_ // TILE_N,),
        in_specs=[
            pl.BlockSpec((TILE_N, 32), lambda i: (i, 0)),
            pl.BlockSpec((TILE_N, 1), lambda i: (i, 0)),
            _full((1, NGRAPH)),
            _full((32, 8)),
        ],
        out_specs=pl.BlockSpec((1, NGRAPH), lambda i: (0, 0)),
        out_shape=jax.ShapeDtypeStruct((1, NGRAPH), jnp.float32),
    )(h1f, batch2d, gsum1, wsi1p)


def _permute_w3(w3, b3):
    w = w3.reshape(RADH, MUL, NSH).transpose(0, 2, 1)
    w = jnp.pad(w, ((0, 0), (0, 0), (0, 32 - MUL))).reshape(RADH, 128)
    b = b3.reshape(MUL, NSH).T
    b = jnp.pad(b, ((0, 0), (0, 32 - MUL))).reshape(1, 128)
    return w, b


def kernel(z, pos, batch, edge_index, emb, W_si0, r0_w1, r0_b1, r0_w2, r0_b2,
           r0_w3, r0_b3, W_si1, r1_w1, r1_b1, r1_w2, r1_b2, r1_w3, r1_b3):
    f32 = jnp.float32
    i32 = jnp.int32

    ptab = jnp.zeros((NP_, 128), f32)
    ptab = ptab.at[:N, :3].set(pos.astype(f32))
    ptab = ptab.at[:N, 3].set(z.astype(f32))
    ptab = ptab.at[:N, 4].set(batch.astype(f32))

    src = edge_index[0].astype(i32)
    dst = edge_index[1].astype(i32)
    srcp = jnp.zeros((EP,), i32).at[:E].set(src)
    dstp = jnp.zeros((EP,), i32).at[:E].set(dst)
    src3d = srcp.reshape(EP // 1024, 8, 128)
    dst3d = dstp.reshape(EP // 1024, 8, 128)
    dstq_3d = (dstp % NACC).reshape(EP // 1024, 8, 128)
    dstc = dstp.reshape(EP, 1)

    emb_pad = jnp.zeros((128, 32), f32).at[:100, :MUL].set(emb.astype(f32))
    wsi0p = jnp.zeros((32, 32), f32).at[:MUL, :MUL].set(W_si0.astype(f32))
    wsi1p = jnp.zeros((32, 8), f32).at[:MUL, 0].set(W_si1[:, 0].astype(f32))
    w3p0, b3p0 = _permute_w3(r0_w3.astype(f32), r0_b3.astype(f32))
    w3p1, b3p1 = _permute_w3(r1_w3.astype(f32), r1_b3.astype(f32))
    b01 = r0_b1.reshape(1, RADH).astype(f32)
    b02 = r0_b2.reshape(1, RADH).astype(f32)
    b11 = r1_b1.reshape(1, RADH).astype(f32)
    b12 = r1_b2.reshape(1, RADH).astype(f32)
    batch2d = jnp.full((NP_, 1), NGRAPH, i32).at[:N, 0].set(batch.astype(i32))
    zacc = jnp.zeros((NACC, 128), f32)

    srcg = _sc_gather_compact(ptab, src3d, 16)
    dstg = _sc_gather_compact(ptab, dst3d, 16)
    msgp, rsh = _tc_edge0(srcg, dstg, dstc, emb_pad, r0_w1.astype(f32), b01,
                          r0_w2.astype(f32), b02, w3p0, b3p0)
    acc0 = _sc_scatter_add(dstq_3d, msgp, zacc)
    h1f, h1t = _tc_node(ptab, acc0, emb_pad, wsi0p)

    xj1g = _sc_gather_compact(h1t, src3d, 32)
    gsum1 = _tc_edge1(rsh, xj1g, dstg, r1_w1.astype(f32), b11,
                      r1_w2.astype(f32), b12, w3p1, b3p1)

    out = _tc_final(h1f, batch2d, gsum1, wsi1p)
    return out.reshape(NGRAPH, 1)
